# Initial kernel scaffold; baseline (speedup 1.0000x reference)
import jax
import jax.numpy as jnp
from jax.experimental import pallas as pl


def kernel(cache_keys, clip_weights, cache_values, res, value_weights, indices):
    raise NotImplementedError("write your pallas kernel here")


# speedup vs baseline: 2.1472x; 2.1472x over previous
import jax
import jax.numpy as jnp
from jax import lax
from jax.experimental import pallas as pl
from jax.experimental.pallas import tpu as pltpu

CATE_NUM = 1000
SHOTS = 16
FEAT_DIM = 1024
FEAT_NUM = 512

CB = 40


def _to_f16(x):
    bits = jax.lax.bitcast_convert_type(x, jnp.int32)
    sign16 = jax.lax.shift_right_logical(bits, 16) & 0x8000
    absb = bits & 0x7FFFFFFF
    e = jax.lax.shift_right_logical(absb, 23)
    base = ((e - 112) << 10) | (jax.lax.shift_right_logical(absb, 13) & 0x3FF)
    lsb = jax.lax.shift_right_logical(absb, 13) & 1
    rnd = jax.lax.shift_right_logical((absb & 0x1FFF) + 0x0FFF + lsb, 13)
    h = base + rnd
    h = jnp.where(e < 113, 0, h)
    h = jnp.where(e > 142, 0x7C00, h)
    return (sign16 | h).astype(jnp.uint16)


def _fused_body(idx_ref, res_ref, res_blk_ref, clip_ref, ck_ref, cv_ref, vw_ref,
                out1_ref, out2_ref, out3_ref, p_scr):
    i = pl.program_id(0)

    @pl.when(i == 0)
    def _():
        idx = idx_ref[...]
        d_iota = lax.broadcasted_iota(jnp.int32, (FEAT_DIM, FEAT_NUM), 0)
        p = (d_iota == idx).astype(jnp.float32)
        p_scr[...] = p
        prod = lax.dot_general(p, res_ref[...], (((1,), (1,)), ((), ())),
                               preferred_element_type=jnp.float32)
        out2_ref[...] = _to_f16(clip_ref[...] + prod)

    res_blk = res_blk_ref[...]
    rf = lax.dot_general(res_blk, p_scr[...], (((1,), (1,)), ((), ())),
                         preferred_element_type=jnp.float32)
    out1_ref[...] = _to_f16(ck_ref[...] + rf[:, None, :])
    out3_ref[...] = _to_f16(cv_ref[...] * vw_ref[...])


def kernel(cache_keys, clip_weights, cache_values, res, value_weights, indices):
    ck3 = cache_keys.reshape(CATE_NUM, SHOTS, FEAT_DIM)
    cv3 = cache_values.reshape(CATE_NUM, SHOTS, CATE_NUM)
    vw3 = value_weights.reshape(CATE_NUM, SHOTS, 1)
    idx2 = indices.reshape(1, FEAT_NUM)

    grid = (CATE_NUM // CB,)
    out1, out2, out3 = pl.pallas_call(
        _fused_body,
        grid=grid,
        in_specs=[
            pl.BlockSpec((1, FEAT_NUM), lambda i: (0, 0)),
            pl.BlockSpec((CATE_NUM, FEAT_NUM), lambda i: (0, 0)),
            pl.BlockSpec((CB, FEAT_NUM), lambda i: (i, 0)),
            pl.BlockSpec((FEAT_DIM, CATE_NUM), lambda i: (0, 0)),
            pl.BlockSpec((CB, SHOTS, FEAT_DIM), lambda i: (i, 0, 0)),
            pl.BlockSpec((CB, SHOTS, CATE_NUM), lambda i: (i, 0, 0)),
            pl.BlockSpec((CB, SHOTS, 1), lambda i: (i, 0, 0)),
        ],
        out_specs=[
            pl.BlockSpec((CB, SHOTS, FEAT_DIM), lambda i: (i, 0, 0)),
            pl.BlockSpec((FEAT_DIM, CATE_NUM), lambda i: (0, 0)),
            pl.BlockSpec((CB, SHOTS, CATE_NUM), lambda i: (i, 0, 0)),
        ],
        out_shape=[
            jax.ShapeDtypeStruct((CATE_NUM, SHOTS, FEAT_DIM), jnp.uint16),
            jax.ShapeDtypeStruct((FEAT_DIM, CATE_NUM), jnp.uint16),
            jax.ShapeDtypeStruct((CATE_NUM, SHOTS, CATE_NUM), jnp.uint16),
        ],
        scratch_shapes=[pltpu.VMEM((FEAT_DIM, FEAT_NUM), jnp.float32)],
    )(idx2, res, res, clip_weights, ck3, cv3, vw3)

    f16 = lambda a: jax.lax.bitcast_convert_type(a, jnp.float16)
    return (f16(out1).reshape(CATE_NUM * SHOTS, FEAT_DIM),
            f16(out2),
            f16(out3).reshape(CATE_NUM * SHOTS, CATE_NUM))
